# baseline (speedup 1.0000x reference)
import numpy as np
import jax
import jax.numpy as jnp
from jax import lax
from jax.experimental import pallas as pl
from jax.experimental.pallas import tpu as pltpu
from jax.experimental.pallas import tpu_sc as plsc

_MASK_BITS = (
    0x6dfda5ef, 0xf7ffb56f, 0xef5bff7f, 0x1edbead9,
    0xfdf7fdfb, 0xaeedb2eb, 0xdbe75ed7, 0x5bffff7c,
    0x7d9aef9b, 0xffbfbffd, 0xcbbfacff, 0xf7bdf6da,
    0x9b7f6dfb, 0xb5b1efbe, 0xb7cb8ebf, 0xbb60d6ff,
    0xbcbcdf7f, 0xf8ff379f, 0x3fddfbfe, 0xcf6ace7f,
    0xd8fff4df, 0xdedeeeef, 0xf7dffcfb, 0xfffdffff,
    0x7b4dffb9, 0xcd6acf7d, 0xd7dddeef, 0xfa7abffb,
    0xf7ed56df, 0xf3fcbf8b, 0x97efe3a8, 0xe3afb96f,
)

_NPLANES = 96
_NPR = 32
_PS = 16
_W = 512
_NBUF = 2
_NC = 2
_NS = 16

_ZERO16 = np.zeros((_PS,), np.float32)


def _sc_mask_body(x_hbm, out_hbm, in_buf, out_buf, in_sem, out_sem):
    wid = lax.axis_index("s") * _NC + lax.axis_index("c")

    bits = jnp.int32(0)
    for r in range(_NPR):
        v = _MASK_BITS[r]
        bits = lax.select(wid == r, jnp.int32(np.int32(np.uint32(v))), bits)
    keep = [lax.eq(lax.shift_right_logical(bits, c) & 1, 0)
            for c in range(_NPR)]

    zero = lax.broadcast_in_dim(jnp.float32(0.0), (_PS,), ())
    for b in range(_NBUF):
        for r in range(_PS):
            for c in range(_NPR):
                out_buf[b, r, pl.ds(c * _PS, _PS)] = zero

    def start_in(b, plane):
        for c in range(_NPR):
            @pl.when(keep[c])
            def _in(c=c):
                sl = pl.ds(c * _PS, _PS)
                pltpu.async_copy(x_hbm.at[plane, wid, :, sl],
                                 in_buf.at[b, :, sl], in_sem.at[b])

    def wait_in(b):
        for c in range(_NPR):
            @pl.when(keep[c])
            def _win(c=c):
                sl = pl.ds(c * _PS, _PS)
                pltpu.make_async_copy(x_hbm.at[0, 0, :, sl],
                                      in_buf.at[b, :, sl],
                                      in_sem.at[b]).wait()

    def start_out(b, plane):
        pltpu.async_copy(out_buf.at[b], out_hbm.at[plane, wid],
                         out_sem.at[b])

    def wait_out(b):
        pltpu.make_async_copy(out_buf.at[b], out_hbm.at[0, 0],
                              out_sem.at[b]).wait()

    for b in range(_NBUF):
        start_in(b, b)

    n_groups = _NPLANES // _NBUF

    def step(g, carry):
        for b in range(_NBUF):
            plane = g * _NBUF + b

            @pl.when(g >= 1)
            def _drain_prev_out(b=b):
                wait_out(b)

            wait_in(b)
            for c in range(_NPR):
                @pl.when(keep[c])
                def _copy(b=b, c=c):
                    sl = pl.ds(c * _PS, _PS)
                    for r in range(_PS):
                        out_buf[b, r, sl] = in_buf[b, r, sl]
            start_out(b, plane)

            @pl.when(g <= n_groups - 2)
            def _prefetch_next(b=b, plane=plane):
                start_in(b, plane + _NBUF)
        return carry

    lax.fori_loop(0, n_groups, step, 0)

    for b in range(_NBUF):
        wait_out(b)


def _masked(x4):
    call = pl.kernel(
        _sc_mask_body,
        out_type=jax.ShapeDtypeStruct((_NPLANES, _NPR, _PS, _W),
                                      jnp.float32),
        mesh=plsc.VectorSubcoreMesh(core_axis_name="c",
                                    subcore_axis_name="s",
                                    num_cores=_NC, num_subcores=_NS),
        scratch_types=[
            pltpu.VMEM((_NBUF, _PS, _W), jnp.float32),
            pltpu.VMEM((_NBUF, _PS, _W), jnp.float32),
            pltpu.SemaphoreType.DMA((_NBUF,)),
            pltpu.SemaphoreType.DMA((_NBUF,)),
        ],
        compiler_params=pltpu.CompilerParams(use_tc_tiling_on_sc=False),
    )
    return call(x4)


def kernel(x):
    x4 = x.reshape(_NPLANES, _NPR, _PS, _W)
    return _masked(x4).reshape(32, 3, 512, 512)


# speedup vs baseline: 4.5887x; 4.5887x over previous
import numpy as np
import jax
import jax.numpy as jnp
from jax import lax
from jax.experimental import pallas as pl
from jax.experimental.pallas import tpu as pltpu
from jax.experimental.pallas import tpu_sc as plsc

_MASK_BITS = (
    0x6dfda5ef, 0xf7ffb56f, 0xef5bff7f, 0x1edbead9,
    0xfdf7fdfb, 0xaeedb2eb, 0xdbe75ed7, 0x5bffff7c,
    0x7d9aef9b, 0xffbfbffd, 0xcbbfacff, 0xf7bdf6da,
    0x9b7f6dfb, 0xb5b1efbe, 0xb7cb8ebf, 0xbb60d6ff,
    0xbcbcdf7f, 0xf8ff379f, 0x3fddfbfe, 0xcf6ace7f,
    0xd8fff4df, 0xdedeeeef, 0xf7dffcfb, 0xfffdffff,
    0x7b4dffb9, 0xcd6acf7d, 0xd7dddeef, 0xfa7abffb,
    0xf7ed56df, 0xf3fcbf8b, 0x97efe3a8, 0xe3afb96f,
)

_NPLANES = 96
_NPR = 32
_PS = 16
_W = 512
_NBUF = 2
_NC = 2
_NS = 16


def _sc_mask_body(x_hbm, out_hbm, in_buf, out_buf, in_sem, out_sem):
    wid = lax.axis_index("s") * _NC + lax.axis_index("c")

    bits = jnp.int32(0)
    for r in range(_NPR):
        bits = lax.select(wid == r, jnp.int32(np.int32(np.uint32(_MASK_BITS[r]))),
                          bits)
    mvecs = []
    for c in range(_NPR):
        keep = lax.eq(lax.shift_right_logical(bits, c) & 1, 0)
        mscal = lax.select(keep, jnp.float32(1.0), jnp.float32(0.0))
        mvecs.append(lax.broadcast_in_dim(mscal, (_PS,), ()))

    def start_in(b, plane):
        pltpu.async_copy(x_hbm.at[plane, wid], in_buf.at[b], in_sem.at[b])

    def wait_in(b):
        pltpu.make_async_copy(x_hbm.at[0, 0], in_buf.at[b],
                              in_sem.at[b]).wait()

    def start_out(b, plane):
        pltpu.async_copy(out_buf.at[b], out_hbm.at[plane, wid],
                         out_sem.at[b])

    def wait_out(b):
        pltpu.make_async_copy(out_buf.at[b], out_hbm.at[0, 0],
                              out_sem.at[b]).wait()

    for b in range(_NBUF):
        start_in(b, b)

    n_groups = _NPLANES // _NBUF

    def step(g, carry):
        for b in range(_NBUF):
            plane = g * _NBUF + b

            @pl.when(g >= 1)
            def _drain_prev_out(b=b):
                wait_out(b)

            wait_in(b)
            for c in range(_NPR):
                sl = pl.ds(c * _PS, _PS)
                for r in range(_PS):
                    out_buf[b, r, sl] = in_buf[b, r, sl] * mvecs[c]
            start_out(b, plane)

            @pl.when(g <= n_groups - 2)
            def _prefetch_next(b=b, plane=plane):
                start_in(b, plane + _NBUF)
        return carry

    lax.fori_loop(0, n_groups, step, 0)

    for b in range(_NBUF):
        wait_out(b)


def _masked(x4):
    call = pl.kernel(
        _sc_mask_body,
        out_type=jax.ShapeDtypeStruct((_NPLANES, _NPR, _PS, _W),
                                      jnp.float32),
        mesh=plsc.VectorSubcoreMesh(core_axis_name="c",
                                    subcore_axis_name="s",
                                    num_cores=_NC, num_subcores=_NS),
        scratch_types=[
            pltpu.VMEM((_NBUF, _PS, _W), jnp.float32),
            pltpu.VMEM((_NBUF, _PS, _W), jnp.float32),
            pltpu.SemaphoreType.DMA((_NBUF,)),
            pltpu.SemaphoreType.DMA((_NBUF,)),
        ],
    )
    return call(x4)


def kernel(x):
    x4 = x.reshape(_NPLANES, _NPR, _PS, _W)
    return _masked(x4).reshape(32, 3, 512, 512)
